# Initial kernel scaffold; baseline (speedup 1.0000x reference)
import jax
import jax.numpy as jnp
from jax.experimental import pallas as pl


def kernel(h, edge_index, fc1, attn1, fc2, attn2):
    raise NotImplementedError("write your pallas kernel here")


# speedup vs baseline: 49.3766x; 49.3766x over previous
import functools

import jax
import jax.numpy as jnp
from jax import lax
from jax.experimental import pallas as pl
from jax.experimental.pallas import tpu as pltpu
from jax.experimental.pallas import tpu_sc as plsc

_N = 512
_IN = 128
_HID = 64
_HEADS = 4
_OUT = 128
_E = 8192
_NEG = -9e15


@functools.lru_cache(maxsize=None)
def _make_edge_scatter(H):
    info = plsc.get_sparse_core_info()
    NC, NS, L = info.num_cores, info.num_subcores, info.num_lanes
    NW = NC * NS
    ROWS = _N // NW
    CHUNKS = _E // L

    @functools.partial(
        pl.kernel,
        out_type=jax.ShapeDtypeStruct((H, _N * _N), jnp.float32),
        mesh=plsc.VectorSubcoreMesh(core_axis_name="c", subcore_axis_name="s"),
        compiler_params=pltpu.CompilerParams(needs_layout_passes=False),
        scratch_types=[
            pltpu.VMEM((_E,), jnp.int32),
            pltpu.VMEM((_E,), jnp.int32),
            pltpu.VMEM((H * _N,), jnp.float32),
            pltpu.VMEM((H * _N,), jnp.float32),
            pltpu.VMEM((H * L,), jnp.float32),
            pltpu.VMEM((H * ROWS * _N,), jnp.float32),
        ],
    )
    def edge_scatter(s1_hbm, s2_hbm, src_hbm, dst_hbm, c_hbm, out_hbm,
                     src_v, dst_v, s1_v, s2_v, c_v, blk_v):
        wid = lax.axis_index("s") * NC + lax.axis_index("c")
        base = wid * ROWS
        pltpu.sync_copy(src_hbm, src_v)
        pltpu.sync_copy(dst_hbm, dst_v)
        pltpu.sync_copy(s1_hbm, s1_v)
        pltpu.sync_copy(s2_hbm, s2_v)
        pltpu.sync_copy(c_hbm, c_v)

        for h in range(H):
            cvec = c_v[pl.ds(h * L, L)]

            def fill(i, _, h=h, cvec=cvec):
                blk_v[pl.ds(h * ROWS * _N + i * L, L)] = cvec
                return 0

            lax.fori_loop(0, ROWS * _N // L, fill, 0)

        def body(e, _):
            sv = src_v[pl.ds(e * L, L)]
            dv = dst_v[pl.ds(e * L, L)]
            local = sv - base
            mask = (local >= 0) & (local < ROWS)
            flat = jnp.where(mask, local * _N + dv, 0)
            for h in range(H):
                v1 = plsc.load_gather(s1_v, [sv + (h * _N)])
                v2 = plsc.load_gather(s2_v, [dv + (h * _N)])
                val = v1 + v2
                val = jnp.where(val >= 0.0, val, 0.2 * val)
                plsc.store_scatter(blk_v, [flat + (h * ROWS * _N)], val,
                                   mask=mask)
            return 0

        lax.fori_loop(0, CHUNKS, body, 0)

        for h in range(H):
            pltpu.sync_copy(blk_v.at[pl.ds(h * ROWS * _N, ROWS * _N)],
                            out_hbm.at[h, pl.ds(base * _N, ROWS * _N)])

    return edge_scatter


def _pre_stage(x, wcat, a1blk, a2blk):
    n, dh = x.shape[0], wcat.shape[1]

    def kern(x_ref, w_ref, a1_ref, a2_ref, wh_ref, s1_ref, s2_ref):
        wh = jnp.dot(x_ref[:], w_ref[:], preferred_element_type=jnp.float32)
        wh_ref[:] = wh
        s1_ref[:] = jnp.dot(wh, a1_ref[:], preferred_element_type=jnp.float32)
        s2_ref[:] = jnp.dot(wh, a2_ref[:], preferred_element_type=jnp.float32)

    return pl.pallas_call(
        kern,
        out_shape=(
            jax.ShapeDtypeStruct((n, dh), jnp.float32),
            jax.ShapeDtypeStruct((n, 128), jnp.float32),
            jax.ShapeDtypeStruct((n, 128), jnp.float32),
        ),
    )(x, wcat, a1blk, a2blk)


def _post_stage(logits, wh, H, d, double_elu):
    def kern(l_ref, wh_ref, o_ref):
        l = l_ref[0]
        m = jnp.max(l, axis=1, keepdims=True)
        p = jnp.exp(l - m)
        s = jnp.sum(p, axis=1, keepdims=True)
        o = jnp.dot(p / s, wh_ref[0], preferred_element_type=jnp.float32)
        o = jnp.where(o > 0, o, jnp.exp(o) - 1.0)
        if double_elu:
            o = jnp.where(o > 0, o, jnp.exp(o) - 1.0)
        o_ref[0] = o

    wh_h = wh.reshape(_N, H, d).transpose(1, 0, 2)
    out = pl.pallas_call(
        kern,
        grid=(H,),
        in_specs=[
            pl.BlockSpec((1, _N, _N), lambda i: (i, 0, 0)),
            pl.BlockSpec((1, _N, d), lambda i: (i, 0, 0)),
        ],
        out_specs=pl.BlockSpec((1, _N, d), lambda i: (i, 0, 0)),
        out_shape=jax.ShapeDtypeStruct((H, _N, d), jnp.float32),
    )(logits, wh_h)
    return out.transpose(1, 0, 2).reshape(_N, H * d)


def _blockdiag(a):
    H, d = a.shape
    cols = jnp.arange(H * d) // d
    sel = (jnp.arange(128)[None, :] == cols[:, None]).astype(a.dtype)
    return sel * a.reshape(-1)[:, None]


def _nonedge_const(attn, L=16):
    s = _NEG * jnp.sum(attn.reshape(attn.shape[0], -1), axis=1)
    c = jnp.where(s >= 0, s, 0.2 * s)
    return jnp.broadcast_to(c[:, None], (attn.shape[0], L)).astype(jnp.float32)


def _gat_layer(x, wcat, a1, a2, cvals, src, dst, H, d):
    wh, s1p, s2p = _pre_stage(x, wcat, _blockdiag(a1), _blockdiag(a2))
    s1 = s1p[:, :H].T.reshape(-1)
    s2 = s2p[:, :H].T.reshape(-1)
    logits = _make_edge_scatter(H)(s1, s2, src, dst, cvals.reshape(-1))
    return logits.reshape(H, _N, _N), wh


def kernel(h, edge_index, fc1, attn1, fc2, attn2):
    src = edge_index[0].astype(jnp.int32)
    dst = edge_index[1].astype(jnp.int32)

    wcat1 = fc1.reshape(_HEADS * _HID, _IN).T
    a1_1 = attn1[:, 0, :_HID]
    a2_1 = attn1[:, 0, _HID:]
    logits1, wh1 = _gat_layer(h, wcat1, a1_1, a2_1, _nonedge_const(attn1),
                              src, dst, _HEADS, _HID)
    out1 = _post_stage(logits1, wh1, _HEADS, _HID, double_elu=False)

    wcat2 = fc2.T
    a1_2 = attn2[:, :_OUT]
    a2_2 = attn2[:, _OUT:]
    logits2, wh2 = _gat_layer(out1, wcat2, a1_2, a2_2, _nonedge_const(attn2),
                              src, dst, 1, _OUT)
    return _post_stage(logits2, wh2, 1, _OUT, double_elu=True)


# speedup vs baseline: 66.3308x; 1.3434x over previous
import functools

import jax
import jax.numpy as jnp
from jax import lax
from jax.experimental import pallas as pl
from jax.experimental.pallas import tpu as pltpu
from jax.experimental.pallas import tpu_sc as plsc

_N = 512
_IN = 128
_HID = 64
_HEADS = 4
_OUT = 128
_E = 8192
_NEG = -9e15


@functools.lru_cache(maxsize=None)
def _make_edge_scatter(H):
    info = plsc.get_sparse_core_info()
    NC, NS, L = info.num_cores, info.num_subcores, info.num_lanes
    NW = NC * NS
    GROUPS = NW // H
    ROWS = _N // GROUPS
    CHUNKS = _E // L
    EU = 4
    FU = 8

    @functools.partial(
        pl.kernel,
        out_type=jax.ShapeDtypeStruct((H * _N * _N,), jnp.float32),
        mesh=plsc.VectorSubcoreMesh(core_axis_name="c", subcore_axis_name="s"),
        compiler_params=pltpu.CompilerParams(needs_layout_passes=False),
        scratch_types=[
            pltpu.VMEM((_E,), jnp.int32),
            pltpu.VMEM((_E,), jnp.int32),
            pltpu.VMEM((H * _N,), jnp.float32),
            pltpu.VMEM((H * _N,), jnp.float32),
            pltpu.VMEM((H * L,), jnp.float32),
            pltpu.VMEM((ROWS * _N,), jnp.float32),
        ],
    )
    def edge_scatter(s1_hbm, s2_hbm, src_hbm, dst_hbm, c_hbm, out_hbm,
                     src_v, dst_v, s1_v, s2_v, c_v, blk_v):
        wid = lax.axis_index("s") * NC + lax.axis_index("c")
        head = wid // GROUPS
        hoff = head * _N
        base = (wid % GROUPS) * ROWS
        pltpu.sync_copy(src_hbm, src_v)
        pltpu.sync_copy(dst_hbm, dst_v)
        pltpu.sync_copy(s1_hbm, s1_v)
        pltpu.sync_copy(s2_hbm, s2_v)
        pltpu.sync_copy(c_hbm, c_v)

        cvec = c_v[pl.ds(head * L, L)]

        def fill(i, _):
            for u in range(FU):
                blk_v[pl.ds((i * FU + u) * L, L)] = cvec
            return 0

        lax.fori_loop(0, ROWS * _N // (L * FU), fill, 0)

        def body(i, _):
            for u in range(EU):
                e = i * EU + u
                sv = src_v[pl.ds(e * L, L)]
                dv = dst_v[pl.ds(e * L, L)]
                local = sv - base
                mask = (local >= 0) & (local < ROWS)
                flat = jnp.where(mask, local * _N + dv, 0)
                v1 = plsc.load_gather(s1_v, [sv + hoff])
                v2 = plsc.load_gather(s2_v, [dv + hoff])
                val = v1 + v2
                val = jnp.where(val >= 0.0, val, 0.2 * val)
                plsc.store_scatter(blk_v, [flat], val, mask=mask)
            return 0

        lax.fori_loop(0, CHUNKS // EU, body, 0)

        pltpu.sync_copy(blk_v,
                        out_hbm.at[pl.ds(head * (_N * _N) + base * _N,
                                         ROWS * _N)])

    return edge_scatter


def _softmax_matmul(l, wh):
    m = jnp.max(l, axis=1, keepdims=True)
    p = jnp.exp(l - m)
    s = jnp.sum(p, axis=1, keepdims=True)
    o = jnp.dot(p / s, wh, preferred_element_type=jnp.float32)
    return jnp.where(o > 0, o, jnp.exp(o) - 1.0)


def _pre_stage(x, wcat, a1blk, a2blk):
    n, dh = x.shape[0], wcat.shape[1]

    def kern(x_ref, w_ref, a1_ref, a2_ref, wh_ref, s1_ref, s2_ref):
        wh = jnp.dot(x_ref[:], w_ref[:], preferred_element_type=jnp.float32)
        wh_ref[:] = wh
        s1_ref[:] = jnp.dot(wh, a1_ref[:], preferred_element_type=jnp.float32)
        s2_ref[:] = jnp.dot(wh, a2_ref[:], preferred_element_type=jnp.float32)

    return pl.pallas_call(
        kern,
        out_shape=(
            jax.ShapeDtypeStruct((n, dh), jnp.float32),
            jax.ShapeDtypeStruct((n, 128), jnp.float32),
            jax.ShapeDtypeStruct((n, 128), jnp.float32),
        ),
    )(x, wcat, a1blk, a2blk)


def _mid_stage(logits1, wh1, wcat2, a1blk2, a2blk2):
    wh_h = wh1.reshape(_N, _HEADS, _HID).transpose(1, 0, 2)

    def kern(l_ref, wh_ref, w2_ref, a1_ref, a2_ref, wh2_ref, s1_ref, s2_ref):
        outs = [_softmax_matmul(l_ref[h], wh_ref[h]) for h in range(_HEADS)]
        out1 = jnp.concatenate(outs, axis=1)
        wh2 = jnp.dot(out1, w2_ref[:], preferred_element_type=jnp.float32)
        wh2_ref[:] = wh2
        s1_ref[:] = jnp.dot(wh2, a1_ref[:], preferred_element_type=jnp.float32)
        s2_ref[:] = jnp.dot(wh2, a2_ref[:], preferred_element_type=jnp.float32)

    return pl.pallas_call(
        kern,
        out_shape=(
            jax.ShapeDtypeStruct((_N, _OUT), jnp.float32),
            jax.ShapeDtypeStruct((_N, 128), jnp.float32),
            jax.ShapeDtypeStruct((_N, 128), jnp.float32),
        ),
    )(logits1.reshape(_HEADS, _N, _N), wh_h, wcat2, a1blk2, a2blk2)


def _final_stage(logits2, wh2):
    def kern(l_ref, wh_ref, o_ref):
        o = _softmax_matmul(l_ref[:], wh_ref[:])
        o_ref[:] = jnp.where(o > 0, o, jnp.exp(o) - 1.0)

    return pl.pallas_call(
        kern,
        out_shape=jax.ShapeDtypeStruct((_N, _OUT), jnp.float32),
    )(logits2.reshape(_N, _N), wh2)


def _blockdiag(a):
    H, d = a.shape
    cols = jnp.arange(H * d) // d
    sel = (jnp.arange(128)[None, :] == cols[:, None]).astype(a.dtype)
    return sel * a.reshape(-1)[:, None]


def _nonedge_const(attn, L=16):
    s = _NEG * jnp.sum(attn.reshape(attn.shape[0], -1), axis=1)
    c = jnp.where(s >= 0, s, 0.2 * s)
    return jnp.broadcast_to(c[:, None], (attn.shape[0], L)
                            ).astype(jnp.float32).reshape(-1)


def _scores_flat(s1p, s2p, H):
    return s1p[:, :H].T.reshape(-1), s2p[:, :H].T.reshape(-1)


def kernel(h, edge_index, fc1, attn1, fc2, attn2):
    src = edge_index[0].astype(jnp.int32)
    dst = edge_index[1].astype(jnp.int32)

    wcat1 = fc1.reshape(_HEADS * _HID, _IN).T
    a1blk2 = _blockdiag(attn2[:, :_OUT])
    a2blk2 = _blockdiag(attn2[:, _OUT:])
    wh1, s1p, s2p = _pre_stage(h, wcat1,
                               _blockdiag(attn1[:, 0, :_HID]),
                               _blockdiag(attn1[:, 0, _HID:]))
    s1, s2 = _scores_flat(s1p, s2p, _HEADS)
    logits1 = _make_edge_scatter(_HEADS)(s1, s2, src, dst,
                                         _nonedge_const(attn1))
    wh2, s1p2, s2p2 = _mid_stage(logits1, wh1, fc2.T, a1blk2, a2blk2)

    s1_2, s2_2 = _scores_flat(s1p2, s2p2, 1)
    logits2 = _make_edge_scatter(1)(s1_2, s2_2, src, dst,
                                    _nonedge_const(attn2))
    return _final_stage(logits2, wh2)


# speedup vs baseline: 94.8758x; 1.4303x over previous
import functools

import jax
import jax.numpy as jnp
from jax import lax
from jax.experimental import pallas as pl
from jax.experimental.pallas import tpu as pltpu
from jax.experimental.pallas import tpu_sc as plsc

_N = 512
_IN = 128
_HID = 64
_HEADS = 4
_OUT = 128
_E = 8192
_NEG = -9e15


@functools.lru_cache(maxsize=None)
def _make_edge_scatter(H):
    info = plsc.get_sparse_core_info()
    NC, NS, L = info.num_cores, info.num_subcores, info.num_lanes
    NW = NC * NS
    GROUPS = NW // H
    ROWS = _N // GROUPS
    CHUNKS = _E // L

    @functools.partial(
        pl.kernel,
        out_type=jax.ShapeDtypeStruct((H, _N, _N), jnp.float32),
        mesh=plsc.VectorSubcoreMesh(core_axis_name="c", subcore_axis_name="s"),
        compiler_params=pltpu.CompilerParams(needs_layout_passes=False),
        scratch_types=[
            pltpu.VMEM((_E,), jnp.int32),
            pltpu.VMEM((_E,), jnp.int32),
            pltpu.VMEM((_N,), jnp.float32),
            pltpu.VMEM((_N,), jnp.float32),
            pltpu.VMEM((L,), jnp.float32),
            pltpu.VMEM((ROWS, _N), jnp.float32),
        ],
    )
    def edge_scatter(s1_hbm, s2_hbm, src_hbm, dst_hbm, c_hbm, out_hbm,
                     src_v, dst_v, s1_v, s2_v, c_v, blk_v):
        wid = lax.axis_index("s") * NC + lax.axis_index("c")
        head = wid // GROUPS
        base = (wid % GROUPS) * ROWS
        pltpu.sync_copy(src_hbm, src_v)
        pltpu.sync_copy(dst_hbm, dst_v)
        pltpu.sync_copy(s1_hbm.at[pl.ds(head * _N, _N)], s1_v)
        pltpu.sync_copy(s2_hbm.at[pl.ds(head * _N, _N)], s2_v)
        pltpu.sync_copy(c_hbm.at[pl.ds(head * L, L)], c_v)

        cvec = c_v[:]

        @plsc.parallel_loop(0, ROWS, 1, unroll=2)
        def _fill(r):
            for j in range(_N // L):
                blk_v[r, pl.ds(j * L, L)] = cvec

        @plsc.parallel_loop(0, CHUNKS, 1, unroll=4)
        def _scan(e):
            sv = src_v[pl.ds(e * L, L)]
            dv = dst_v[pl.ds(e * L, L)]
            local = sv - base
            mask = (local >= 0) & (local < ROWS)
            local = jnp.where(mask, local, 0)
            v1 = plsc.load_gather(s1_v, [sv])
            v2 = plsc.load_gather(s2_v, [dv])
            val = v1 + v2
            val = jnp.where(val >= 0.0, val, 0.2 * val)
            plsc.store_scatter(blk_v, [local, dv], val, mask=mask)

        pltpu.sync_copy(blk_v, out_hbm.at[head, pl.ds(base, ROWS), :])

    return edge_scatter


_CONTRACT_MINOR = (((1,), (1,)), ((), ()))


def _scores_and_const(wh, a_ref, H, d, s1_ref, s2_ref, c_ref):
    for h in range(H):
        whh = wh[:, h * d:(h + 1) * d]
        s1_ref[h, :] = jnp.dot(whh, a_ref[h, :d],
                               preferred_element_type=jnp.float32)
        s2_ref[h, :] = jnp.dot(whh, a_ref[h, d:],
                               preferred_element_type=jnp.float32)
    craw = _NEG * jnp.sum(a_ref[:], axis=1, keepdims=True)
    cv = jnp.where(craw >= 0, craw, 0.2 * craw)
    c_ref[:] = jnp.broadcast_to(cv, (H, 16))


def _softmax_matmul(l, wh):
    m = jnp.max(l, axis=1, keepdims=True)
    p = jnp.exp(l - m)
    s = jnp.sum(p, axis=1, keepdims=True)
    o = jnp.dot(p / s, wh, preferred_element_type=jnp.float32)
    return jnp.where(o > 0, o, jnp.exp(o) - 1.0)


def _pre_stage(x, w, a):
    n = x.shape[0]
    H, twod = a.shape
    d = twod // 2

    def kern(x_ref, w_ref, a_ref, wh_ref, s1_ref, s2_ref, c_ref):
        wh = lax.dot_general(x_ref[:], w_ref[:], _CONTRACT_MINOR,
                             preferred_element_type=jnp.float32)
        wh_ref[:] = wh
        _scores_and_const(wh, a_ref, H, d, s1_ref, s2_ref, c_ref)

    return pl.pallas_call(
        kern,
        out_shape=(
            jax.ShapeDtypeStruct((n, H * d), jnp.float32),
            jax.ShapeDtypeStruct((H, n), jnp.float32),
            jax.ShapeDtypeStruct((H, n), jnp.float32),
            jax.ShapeDtypeStruct((H, 16), jnp.float32),
        ),
    )(x, w, a)


def _mid_stage(logits1, wh1, w2, a2):
    def kern(l_ref, wh_ref, w2_ref, a_ref, wh2_ref, s1_ref, s2_ref, c_ref):
        outs = [_softmax_matmul(l_ref[h],
                                wh_ref[:, h * _HID:(h + 1) * _HID])
                for h in range(_HEADS)]
        out1 = jnp.concatenate(outs, axis=1)
        wh2 = lax.dot_general(out1, w2_ref[:], _CONTRACT_MINOR,
                              preferred_element_type=jnp.float32)
        wh2_ref[:] = wh2
        _scores_and_const(wh2, a_ref, 1, _OUT, s1_ref, s2_ref, c_ref)

    return pl.pallas_call(
        kern,
        out_shape=(
            jax.ShapeDtypeStruct((_N, _OUT), jnp.float32),
            jax.ShapeDtypeStruct((1, _N), jnp.float32),
            jax.ShapeDtypeStruct((1, _N), jnp.float32),
            jax.ShapeDtypeStruct((1, 16), jnp.float32),
        ),
    )(logits1, wh1, w2, a2)


def _final_stage(logits2, wh2):
    def kern(l_ref, wh_ref, o_ref):
        o = _softmax_matmul(l_ref[0], wh_ref[:])
        o_ref[:] = jnp.where(o > 0, o, jnp.exp(o) - 1.0)

    return pl.pallas_call(
        kern,
        out_shape=jax.ShapeDtypeStruct((_N, _OUT), jnp.float32),
    )(logits2, wh2)


def kernel(h, edge_index, fc1, attn1, fc2, attn2):
    src = edge_index[0].astype(jnp.int32)
    dst = edge_index[1].astype(jnp.int32)

    wh1, s1, s2, c1 = _pre_stage(h, fc1.reshape(_HEADS * _HID, _IN),
                                 attn1.reshape(_HEADS, 2 * _HID))
    logits1 = _make_edge_scatter(_HEADS)(s1.reshape(-1), s2.reshape(-1),
                                         src, dst, c1.reshape(-1))
    wh2, s1_2, s2_2, c2 = _mid_stage(logits1, wh1, fc2, attn2)

    logits2 = _make_edge_scatter(1)(s1_2.reshape(-1), s2_2.reshape(-1),
                                    src, dst, c2.reshape(-1))
    return _final_stage(logits2, wh2)
